# Initial kernel scaffold; baseline (speedup 1.0000x reference)
import jax
import jax.numpy as jnp
from jax.experimental import pallas as pl


def kernel(output, subgraph_representation, node_labels):
    raise NotImplementedError("write your pallas kernel here")


# speedup vs baseline: 1.3650x; 1.3650x over previous
import jax
import jax.numpy as jnp
from jax import lax
from jax.experimental import pallas as pl
from jax.experimental.pallas import tpu as pltpu
from jax.experimental.pallas import tpu_sc as plsc

NC = 2
NS = 16
NW = NC * NS

B, N, T, F = 4, 2048, 12, 64
A, K = 4, 32
ROW = T * F
NF = (1 + A) * F

ROWS_PER_W = (B * N) // NW
WPB = N // ROWS_PER_W
CHUNK = 8
NCHUNK = ROWS_PER_W // CHUNK


def _body(src_hbm, table_hbm, lab_hbm, out_hbm, lab_v, idx_all,
          pbuf, gbufs, rowbuf, sem):
    cid = lax.axis_index("c")
    sid = lax.axis_index("s")
    wid = sid * NC + cid
    b = wid // WPB
    n0b = (wid % WPB) * ROWS_PER_W

    pltpu.sync_copy(lab_hbm.at[:, pl.ds(n0b, ROWS_PER_W)], lab_v)

    def build_idx(q, _):
        off = pl.multiple_of(q * 16, 16)
        for a in range(A):
            idx_all[pl.ds(a * ROWS_PER_W + off, 16)] = \
                lab_v[a, pl.ds(off, 16)] + (b * A + a) * K
        return 0
    lax.fori_loop(0, ROWS_PER_W // 16, build_idx, 0)

    def chunk_body(j, _):
        n0 = n0b + j * CHUNK
        pass_cp = pltpu.async_copy(src_hbm.at[pl.ds(b * N + n0, CHUNK)],
                                   pbuf, sem)
        gather_cps = [
            pltpu.async_copy(
                table_hbm.at[idx_all.at[pl.ds(a * ROWS_PER_W + j * CHUNK,
                                              CHUNK)]],
                gbufs.at[a], sem)
            for a in range(A)
        ]
        pass_cp.wait()
        for cp in gather_cps:
            cp.wait()

        def assemble(i, _):
            for t in range(T):
                for u in range(F // 16):
                    rowbuf[i, t, pl.ds(u * 16, 16)] = \
                        pbuf[i, pl.ds(t * F + u * 16, 16)]
                for a in range(A):
                    for u in range(F // 16):
                        rowbuf[i, t, pl.ds((1 + a) * F + u * 16, 16)] = \
                            gbufs[a, i, pl.ds(t * F + u * 16, 16)]
            return 0
        lax.fori_loop(0, CHUNK, assemble, 0)

        pltpu.sync_copy(rowbuf, out_hbm.at[b, pl.ds(n0, CHUNK)])
        return 0

    lax.fori_loop(0, NCHUNK, chunk_body, 0)


def kernel(output, subgraph_representation, node_labels):
    src2 = output.reshape(B * N, ROW)
    table = subgraph_representation.reshape(B * A * K, ROW)
    lab = node_labels.astype(jnp.int32)
    mesh = plsc.VectorSubcoreMesh(
        core_axis_name="c", subcore_axis_name="s",
        num_cores=NC, num_subcores=NS,
    )
    return pl.kernel(
        _body,
        out_type=jax.ShapeDtypeStruct((B, N, T, NF), jnp.float32),
        mesh=mesh,
        scratch_types=[
            pltpu.VMEM((A, ROWS_PER_W), jnp.int32),
            pltpu.VMEM((A * ROWS_PER_W,), jnp.int32),
            pltpu.VMEM((CHUNK, ROW), jnp.float32),
            pltpu.VMEM((A, CHUNK, ROW), jnp.float32),
            pltpu.VMEM((CHUNK, T, NF), jnp.float32),
            pltpu.SemaphoreType.DMA,
        ],
    )(src2, table, lab)


# speedup vs baseline: 1.7335x; 1.2699x over previous
import jax
import jax.numpy as jnp
from jax import lax
from jax.experimental import pallas as pl
from jax.experimental.pallas import tpu as pltpu
from jax.experimental.pallas import tpu_sc as plsc

NC = 2
NS = 16
NW = NC * NS

B, N, T, F = 4, 2048, 12, 64
A, K = 4, 32
ROW = T * F
NF = (1 + A) * F

ROWS_PER_W = (B * N) // NW
WPB = N // ROWS_PER_W
CHUNK = 8
HALF = CHUNK // 2
NCHUNK = ROWS_PER_W // CHUNK


def _body(src_hbm, table_hbm, lab_hbm, out_hbm, lab_v, idx_all,
          pbuf2, gbufs2, rowbuf2, semi0, semi1, semo0, semo1):
    cid = lax.axis_index("c")
    sid = lax.axis_index("s")
    wid = sid * NC + cid
    b = wid // WPB
    n0b = (wid % WPB) * ROWS_PER_W

    pltpu.sync_copy(lab_hbm.at[:, pl.ds(n0b, ROWS_PER_W)], lab_v)

    def build_idx(q, _):
        off = pl.multiple_of(q * 16, 16)
        for a in range(A):
            idx_all[pl.ds(a * ROWS_PER_W + off, 16)] = \
                lab_v[a, pl.ds(off, 16)] + (b * A + a) * K
        return 0
    lax.fori_loop(0, ROWS_PER_W // 16, build_idx, 0)

    semi = (semi0, semi1)
    semo = (semo0, semo1)

    def in_descs(j, p):
        n0 = n0b + j * CHUNK
        descs = [pltpu.make_async_copy(
            src_hbm.at[pl.ds(b * N + n0, CHUNK)], pbuf2.at[p], semi[p])]
        for a in range(A):
            descs.append(pltpu.make_async_copy(
                table_hbm.at[idx_all.at[pl.ds(a * ROWS_PER_W + j * CHUNK,
                                              CHUNK)]],
                gbufs2.at[p, a], semi[p]))
        return descs

    def out_desc(j, h):
        n0 = n0b + j * CHUNK + h * HALF
        return pltpu.make_async_copy(
            rowbuf2.at[h], out_hbm.at[b, pl.ds(n0, HALF)], semo[h])

    def assemble_half(p, h):
        def one_row(i4, _):
            i = h * HALF + i4
            def one_t(t, _):
                base = pl.multiple_of(t * F, 16)
                for u in range(F // 16):
                    rowbuf2[h, i4, t, pl.ds(u * 16, 16)] = \
                        pbuf2[p, i, pl.ds(base + u * 16, 16)]
                for a in range(A):
                    for u in range(F // 16):
                        rowbuf2[h, i4, t, pl.ds((1 + a) * F + u * 16, 16)] = \
                            gbufs2[p, a, i, pl.ds(base + u * 16, 16)]
                return 0
            return lax.fori_loop(0, T, one_t, 0)
        lax.fori_loop(0, HALF, one_row, 0)

    for d in in_descs(0, 0):
        d.start()

    def loop(jj, _):
        for p in (0, 1):
            j = 2 * jj + p
            jn = j + 1

            @pl.when(jn < NCHUNK)
            def _():
                for d in in_descs(jn, 1 - p):
                    d.start()

            for d in in_descs(j, p):
                d.wait()

            for h in (0, 1):
                @pl.when(j > 0)
                def _():
                    out_desc(j, h).wait()
                assemble_half(p, h)
                out_desc(j, h).start()
        return 0

    lax.fori_loop(0, NCHUNK // 2, loop, 0)
    out_desc(NCHUNK - 1, 0).wait()
    out_desc(NCHUNK - 1, 1).wait()


def kernel(output, subgraph_representation, node_labels):
    src2 = output.reshape(B * N, ROW)
    table = subgraph_representation.reshape(B * A * K, ROW)
    lab = node_labels.astype(jnp.int32)
    mesh = plsc.VectorSubcoreMesh(
        core_axis_name="c", subcore_axis_name="s",
        num_cores=NC, num_subcores=NS,
    )
    return pl.kernel(
        _body,
        out_type=jax.ShapeDtypeStruct((B, N, T, NF), jnp.float32),
        mesh=mesh,
        scratch_types=[
            pltpu.VMEM((A, ROWS_PER_W), jnp.int32),
            pltpu.VMEM((A * ROWS_PER_W,), jnp.int32),
            pltpu.VMEM((2, CHUNK, ROW), jnp.float32),
            pltpu.VMEM((2, A, CHUNK, ROW), jnp.float32),
            pltpu.VMEM((2, HALF, T, NF), jnp.float32),
            pltpu.SemaphoreType.DMA,
            pltpu.SemaphoreType.DMA,
            pltpu.SemaphoreType.DMA,
            pltpu.SemaphoreType.DMA,
        ],
    )(src2, table, lab)
